# Initial kernel scaffold; baseline (speedup 1.0000x reference)
import jax
import jax.numpy as jnp
from jax.experimental import pallas as pl


def kernel(boxes, scores, labels):
    raise NotImplementedError("write your pallas kernel here")


# speedup vs baseline: 43.6938x; 43.6938x over previous
import jax
import jax.numpy as jnp
from jax import lax
from jax.experimental import pallas as pl

_PRE_T = 0.05
_NMS_T = 0.6
_K = 1000
_KP = 1024
_RP = 104
_OUT = 100


def _nms_body(pr_ref, pc_ref, bx_ref, out_ref):
    m = jnp.max(bx_ref[...]) + 1.0
    x1r = pr_ref[0:1, :]
    y1r = pr_ref[1:2, :]
    x2r = pr_ref[2:3, :]
    y2r = pr_ref[3:4, :]
    sr = pr_ref[4:5, :]
    lr = pr_ref[5:6, :]
    x1c = pc_ref[:, 0:1]
    y1c = pc_ref[:, 1:2]
    x2c = pc_ref[:, 2:3]
    y2c = pc_ref[:, 3:4]
    sc = pc_ref[:, 4:5]
    lc = pc_ref[:, 5:6]

    offr = lr * m
    offc = lc * m
    ox1r = x1r + offr
    oy1r = y1r + offr
    ox2r = x2r + offr
    oy2r = y2r + offr
    ox1c = x1c + offc
    oy1c = y1c + offc
    ox2c = x2c + offc
    oy2c = y2c + offc

    area_r = (ox2r - ox1r) * (oy2r - oy1r)
    area_c = (ox2c - ox1c) * (oy2c - oy1c)
    w = jnp.maximum(jnp.minimum(ox2c, ox2r) - jnp.maximum(ox1c, ox1r), 0.0)
    h = jnp.maximum(jnp.minimum(oy2c, oy2r) - jnp.maximum(oy1c, oy1r), 0.0)
    inter = w * h
    union = area_c + area_r - inter
    iou = inter / jnp.maximum(union, 1e-6)

    alive_r = sr > _PRE_T
    alive_c = sc > _PRE_T
    hit = (iou > _NMS_T) & alive_r & alive_c
    ri = lax.broadcasted_iota(jnp.int32, (_KP, _KP), 0)
    ci = lax.broadcasted_iota(jnp.int32, (_KP, _KP), 1)
    sup_by_col = hit & (ci < ri)
    sup_by_row = hit & (ri < ci)

    def body(carry):
        k_row_f, _ = carry
        k_row = k_row_f > 0.0
        supc = jnp.any(sup_by_col & k_row, axis=1, keepdims=True)
        k_col = alive_c & jnp.logical_not(supc)
        supr = jnp.any(sup_by_row & k_col, axis=0, keepdims=True)
        k_new = alive_r & jnp.logical_not(supr)
        changed = jnp.any(k_new != k_row)
        return (k_new.astype(jnp.float32), changed)

    k_row_f, _ = lax.while_loop(
        lambda c: c[1], body, (alive_r.astype(jnp.float32), jnp.bool_(True)))
    k_row = k_row_f > 0.0
    supc = jnp.any(sup_by_col & k_row, axis=1, keepdims=True)
    k_col = alive_c & jnp.logical_not(supc)

    ltf = (ri < ci).astype(jnp.float32)
    kcf = k_col.astype(jnp.float32)
    kept_before = jnp.sum(kcf * ltf, axis=0, keepdims=True)
    nk_before = jnp.sum((1.0 - kcf) * ltf, axis=0, keepdims=True)
    total_kept = jnp.sum(kcf)
    slot = jnp.where(k_row, kept_before, total_kept + nk_before)

    ro = lax.broadcasted_iota(jnp.int32, (_RP, _KP), 0)
    onehot = (ro == slot.astype(jnp.int32)).astype(jnp.float32)
    s_out = jnp.where(k_row, sr, -1.0)

    lane = lax.broadcasted_iota(jnp.int32, (_RP, 128), 1)
    acc = jnp.zeros((_RP, 128), jnp.float32)
    for c, v in enumerate((x1r, y1r, x2r, y2r, s_out)):
        colv = jnp.sum(onehot * v, axis=1, keepdims=True)
        acc = acc + jnp.where(lane == c, colv, 0.0)
    out_ref[...] = acc


def kernel(boxes, scores, labels):
    s = jnp.where(scores > _PRE_T, scores, -1.0)
    top_s, top_i = lax.top_k(s, _K)
    top_b = jnp.take(boxes, top_i, axis=0)
    top_l = jnp.take(labels, top_i, axis=0).astype(jnp.float32)
    rows = jnp.concatenate(
        [top_b.T, top_s[None, :], top_l[None, :]], axis=0)
    pr = jnp.pad(rows, ((0, 0), (0, _KP - _K)))
    pc = pr.T
    bx = boxes.reshape(625, 128)
    out = pl.pallas_call(
        _nms_body,
        out_shape=jax.ShapeDtypeStruct((_RP, 128), jnp.float32),
    )(pr, pc, bx)
    return out[:_OUT, :5]
